# baseline (speedup 1.0000x reference)
import jax
import jax.numpy as jnp
from jax.experimental import pallas as pl
from jax.experimental.pallas import tpu as pltpu

_TOKENS = 32768
_HIDDEN = 4096
_EXPERTS = 64
_CH = 512
_NCH = _TOKENS // _CH
_NBUF = 6


def _router_body(hs_hbm, w_ref, probsT_hbm, sel_ref, buf, sems, osb, osems):
    def chunk_copy(i, slot):
        return pltpu.make_async_copy(
            hs_hbm.at[pl.ds(i * _CH, _CH), :], buf.at[slot], sems.at[slot])

    def out_copy(i, slot):
        return pltpu.make_async_copy(
            osb.at[slot], probsT_hbm.at[:, pl.ds(i * _CH, _CH)],
            osems.at[slot])

    for k in range(_NBUF - 1):
        chunk_copy(k, k).start()

    def step(i, carry):
        slot = jax.lax.rem(i, _NBUF)
        chunk_copy(i, slot).wait()

        nxt = i + _NBUF - 1
        @pl.when(nxt < _NCH)
        def _():
            chunk_copy(nxt, jax.lax.rem(nxt, _NBUF)).start()

        logits = jax.lax.dot_general(
            w_ref[:], buf[slot], (((1,), (1,)), ((), ())),
            preferred_element_type=jnp.float32)
        m = jnp.max(logits, axis=0, keepdims=True)
        e = jnp.exp(logits - m)
        probs = e / jnp.sum(e, axis=0, keepdims=True)

        oslot = jax.lax.rem(i, 2)
        @pl.when(i >= 2)
        def _():
            out_copy(i - 2, oslot).wait()
        osb[oslot] = probs
        out_copy(i, oslot).start()

        pm = jnp.max(probs, axis=0, keepdims=True)
        idx = jax.lax.broadcasted_iota(jnp.int32, probs.shape, 0)
        sel = jnp.min(jnp.where(probs == pm, idx, _EXPERTS), axis=0)
        sel_ref[pl.ds(i * _CH, _CH)] = sel
        return carry

    jax.lax.fori_loop(0, _NCH, step, 0)
    out_copy(_NCH - 2, jax.lax.rem(_NCH - 2, 2)).wait()
    out_copy(_NCH - 1, jax.lax.rem(_NCH - 1, 2)).wait()


def kernel(hidden_states, W):
    probs_t, sel = pl.pallas_call(
        _router_body,
        in_specs=[
            pl.BlockSpec(memory_space=pltpu.HBM),
            pl.BlockSpec(memory_space=pltpu.VMEM),
        ],
        out_specs=[
            pl.BlockSpec(memory_space=pltpu.HBM),
            pl.BlockSpec(memory_space=pltpu.VMEM),
        ],
        out_shape=[
            jax.ShapeDtypeStruct((_EXPERTS, _TOKENS), jnp.float32),
            jax.ShapeDtypeStruct((_TOKENS,), jnp.int32),
        ],
        scratch_shapes=[
            pltpu.VMEM((_NBUF, _CH, _HIDDEN), jnp.float32),
            pltpu.SemaphoreType.DMA((_NBUF,)),
            pltpu.VMEM((2, _EXPERTS, _CH), jnp.float32),
            pltpu.SemaphoreType.DMA((2,)),
        ],
        compiler_params=pltpu.CompilerParams(
            vmem_limit_bytes=100 * 1024 * 1024,
            skip_device_barrier=True,
        ),
    )(hidden_states, W)
    return probs_t.T, sel


# speedup vs baseline: 1.0131x; 1.0131x over previous
import jax
import jax.numpy as jnp
from jax.experimental import pallas as pl
from jax.experimental.pallas import tpu as pltpu

_TOKENS = 32768
_HIDDEN = 4096
_EXPERTS = 64
_CH = 1024
_NCH = _TOKENS // _CH
_NBUF = 3


def _router_body(hs_hbm, w_ref, probsT_hbm, sel_ref, buf, sems, osb, osems):
    def chunk_copy(i, slot):
        return pltpu.make_async_copy(
            hs_hbm.at[pl.ds(i * _CH, _CH), :], buf.at[slot], sems.at[slot])

    def out_copy(i, slot):
        return pltpu.make_async_copy(
            osb.at[slot], probsT_hbm.at[:, pl.ds(i * _CH, _CH)],
            osems.at[slot])

    for k in range(_NBUF - 1):
        chunk_copy(k, k).start()

    def step(i, carry):
        slot = jax.lax.rem(i, _NBUF)
        chunk_copy(i, slot).wait()

        nxt = i + _NBUF - 1
        @pl.when(nxt < _NCH)
        def _():
            chunk_copy(nxt, jax.lax.rem(nxt, _NBUF)).start()

        logits = jax.lax.dot_general(
            w_ref[:], buf[slot], (((1,), (1,)), ((), ())),
            preferred_element_type=jnp.float32)
        m = jnp.max(logits, axis=0, keepdims=True)
        e = jnp.exp(logits - m)
        probs = e / jnp.sum(e, axis=0, keepdims=True)

        oslot = jax.lax.rem(i, 2)
        @pl.when(i >= 2)
        def _():
            out_copy(i - 2, oslot).wait()
        osb[oslot] = probs
        out_copy(i, oslot).start()

        pm = jnp.max(probs, axis=0, keepdims=True)
        idx = jax.lax.broadcasted_iota(jnp.int32, probs.shape, 0)
        sel = jnp.min(jnp.where(probs == pm, idx, _EXPERTS), axis=0)
        sel_ref[pl.ds(i * _CH, _CH)] = sel
        return carry

    jax.lax.fori_loop(0, _NCH, step, 0)
    out_copy(_NCH - 2, jax.lax.rem(_NCH - 2, 2)).wait()
    out_copy(_NCH - 1, jax.lax.rem(_NCH - 1, 2)).wait()


def kernel(hidden_states, W):
    probs_t, sel = pl.pallas_call(
        _router_body,
        in_specs=[
            pl.BlockSpec(memory_space=pltpu.HBM),
            pl.BlockSpec(memory_space=pltpu.VMEM),
        ],
        out_specs=[
            pl.BlockSpec(memory_space=pltpu.HBM),
            pl.BlockSpec(memory_space=pltpu.VMEM),
        ],
        out_shape=[
            jax.ShapeDtypeStruct((_EXPERTS, _TOKENS), jnp.float32),
            jax.ShapeDtypeStruct((_TOKENS,), jnp.int32),
        ],
        scratch_shapes=[
            pltpu.VMEM((_NBUF, _CH, _HIDDEN), jnp.float32),
            pltpu.SemaphoreType.DMA((_NBUF,)),
            pltpu.VMEM((2, _EXPERTS, _CH), jnp.float32),
            pltpu.SemaphoreType.DMA((2,)),
        ],
        compiler_params=pltpu.CompilerParams(
            vmem_limit_bytes=100 * 1024 * 1024,
            skip_device_barrier=True,
        ),
    )(hidden_states, W)
    return probs_t.T, sel
